# Initial kernel scaffold; baseline (speedup 1.0000x reference)
import jax
import jax.numpy as jnp
from jax.experimental import pallas as pl


def kernel(x, W):
    raise NotImplementedError("write your pallas kernel here")


# speedup vs baseline: 1.0120x; 1.0120x over previous
import functools

import jax
import jax.numpy as jnp
from jax import lax
from jax.experimental import pallas as pl
from jax.experimental.pallas import tpu as pltpu
from jax.experimental.pallas import tpu_sc as plsc

K = 8192
D = 256
T = 16384
TM = 256
TN = 2048
NCHUNK = K // TN
GRID = T // TM

NW = 32
BPW = T // NW
CH = 128
NCH = BPW // CH


def _dist_argmin_body(x_ref, wt_ref, x2_ref, w2_ref, idx_ref):
    x = x_ref[...]
    x2 = x2_ref[...]
    xb = x.astype(jnp.bfloat16)

    best = jnp.full((TM, 1), jnp.inf, dtype=jnp.float32)
    besti = jnp.zeros((TM, 1), dtype=jnp.int32)
    for c in range(NCHUNK):
        w_c = wt_ref[c * TN:(c + 1) * TN, :].astype(jnp.bfloat16)
        mm = lax.dot_general(xb, w_c, (((1,), (1,)), ((), ())),
                             preferred_element_type=jnp.float32)
        w2_c = w2_ref[:, c * TN:(c + 1) * TN]
        d = (x2 + w2_c) - 2.0 * mm
        cmin = jnp.min(d, axis=1, keepdims=True)
        jj = lax.broadcasted_iota(jnp.int32, (TM, TN), 1) + c * TN
        cidx = jnp.min(jnp.where(d == cmin, jj, jnp.int32(2**31 - 1)),
                       axis=1, keepdims=True)
        upd = cmin < best
        besti = jnp.where(upd, cidx, besti)
        best = jnp.where(upd, cmin, best)
    idx_ref[...] = besti


def _dist_argmin(xf, w, x2, w2r):
    return pl.pallas_call(
        _dist_argmin_body,
        grid=(GRID,),
        in_specs=[
            pl.BlockSpec((TM, D), lambda i: (i, 0)),
            pl.BlockSpec((K, D), lambda i: (0, 0)),
            pl.BlockSpec((TM, 1), lambda i: (i, 0)),
            pl.BlockSpec((1, K), lambda i: (0, 0)),
        ],
        out_specs=pl.BlockSpec((TM, 1), lambda i: (i, 0)),
        out_shape=jax.ShapeDtypeStruct((T, 1), jnp.int32),
    )(xf, w, x2, w2r)


def _sc_gather(w, idx3):
    mesh = plsc.VectorSubcoreMesh(core_axis_name="c", subcore_axis_name="s")

    @functools.partial(
        pl.kernel,
        out_type=jax.ShapeDtypeStruct((T, D), jnp.float32),
        mesh=mesh,
        scratch_types=[
            pltpu.VMEM((CH,), jnp.int32),
            pltpu.VMEM((CH,), jnp.int32),
            pltpu.VMEM((CH, D), jnp.float32),
            pltpu.VMEM((CH, D), jnp.float32),
            pltpu.SemaphoreType.DMA,
            pltpu.SemaphoreType.DMA,
        ],
    )
    def gather_kernel(w_hbm, idx_hbm, out_hbm, idx_a, idx_b, buf_a, buf_b,
                      sem_a, sem_b):
        wid = lax.axis_index("s") * 2 + lax.axis_index("c")
        base = wid * BPW
        idxv = [idx_a, idx_b]
        bufv = [buf_a, buf_b]
        semv = [sem_a, sem_b]
        copies = [None, None]
        pltpu.sync_copy(idx_hbm.at[wid, 0], idx_a)
        copies[0] = pltpu.async_copy(w_hbm.at[idx_a], buf_a, sem_a)
        for c in range(NCH):
            cur = c % 2
            nxt = (c + 1) % 2
            if c + 1 < NCH:
                pltpu.sync_copy(idx_hbm.at[wid, c + 1], idxv[nxt])
                copies[nxt] = pltpu.async_copy(w_hbm.at[idxv[nxt]],
                                               bufv[nxt], semv[nxt])
            copies[cur].wait()
            pltpu.sync_copy(bufv[cur], out_hbm.at[pl.ds(base + c * CH, CH)])

    return gather_kernel(w, idx3)


def _st_loss_body(x_ref, q_ref, st_ref, loss_ref):
    i = pl.program_id(0)
    x = x_ref[...]
    q = q_ref[...]
    diff = q - x
    st_ref[...] = x + diff
    part = jnp.sum(diff * diff)
    prev = jnp.where(i == 0, 0.0, loss_ref[0, 0])
    acc = prev + part
    scale = jnp.where(i == GRID - 1, jnp.float32(1.25 / (T * D)), 1.0)
    loss_ref[...] = jnp.reshape(acc * scale, (1, 1))


def _st_loss(xf, q):
    return pl.pallas_call(
        _st_loss_body,
        grid=(GRID,),
        in_specs=[
            pl.BlockSpec((TM, D), lambda i: (i, 0)),
            pl.BlockSpec((TM, D), lambda i: (i, 0)),
        ],
        out_specs=[
            pl.BlockSpec((TM, D), lambda i: (i, 0)),
            pl.BlockSpec((1, 1), lambda i: (0, 0)),
        ],
        out_shape=[
            jax.ShapeDtypeStruct((T, D), jnp.float32),
            jax.ShapeDtypeStruct((1, 1), jnp.float32),
        ],
    )(xf, q)


def kernel(x, W):
    B, S, _ = x.shape
    xf = x.reshape(T, D)
    x2 = jnp.sum(xf ** 2, axis=1, keepdims=True)
    w2r = jnp.sum(W ** 2, axis=1).reshape(1, K)
    idx = _dist_argmin(xf, W, x2, w2r)
    idx_flat = idx.reshape(T)
    q = _sc_gather(W, idx.reshape(NW, NCH, CH))
    st, loss2 = _st_loss(xf, q)
    return (st.reshape(B, S, D), loss2[0, 0], idx_flat.reshape(B, S))


# speedup vs baseline: 1.1717x; 1.1577x over previous
import functools

import jax
import jax.numpy as jnp
from jax import lax
from jax.experimental import pallas as pl
from jax.experimental.pallas import tpu as pltpu
from jax.experimental.pallas import tpu_sc as plsc

K = 8192
D = 256
T = 16384
TM = 256
TN = 2048
NCHUNK = K // TN
GRID = T // TM

NW = 32
BPW = T // NW
CH = 128
NCH = BPW // CH


def _dist_argmin_body(x_ref, wt_ref, x2_ref, w2_ref, idx_ref):
    x = x_ref[...]
    x2 = x2_ref[...]
    xb = x.astype(jnp.bfloat16)

    wb = wt_ref[...].astype(jnp.bfloat16)
    mm = lax.dot_general(xb, wb, (((1,), (1,)), ((), ())),
                         preferred_element_type=jnp.float32)
    d = (x2 + w2_ref[...]) - 2.0 * mm
    idx_ref[...] = jnp.argmin(d, axis=1).astype(jnp.int32).reshape(TM, 1)


def _dist_argmin(xf, w, x2, w2r):
    return pl.pallas_call(
        _dist_argmin_body,
        grid=(GRID,),
        in_specs=[
            pl.BlockSpec((TM, D), lambda i: (i, 0)),
            pl.BlockSpec((K, D), lambda i: (0, 0)),
            pl.BlockSpec((TM, 1), lambda i: (i, 0)),
            pl.BlockSpec((1, K), lambda i: (0, 0)),
        ],
        out_specs=pl.BlockSpec((TM, 1), lambda i: (i, 0)),
        out_shape=jax.ShapeDtypeStruct((T, 1), jnp.int32),
    )(xf, w, x2, w2r)


def _sc_gather(w, idx3):
    mesh = plsc.VectorSubcoreMesh(core_axis_name="c", subcore_axis_name="s")

    @functools.partial(
        pl.kernel,
        out_type=jax.ShapeDtypeStruct((T, D), jnp.float32),
        mesh=mesh,
        scratch_types=[
            pltpu.VMEM((CH,), jnp.int32),
            pltpu.VMEM((CH,), jnp.int32),
            pltpu.VMEM((CH, D), jnp.float32),
            pltpu.VMEM((CH, D), jnp.float32),
            pltpu.SemaphoreType.DMA,
            pltpu.SemaphoreType.DMA,
        ],
    )
    def gather_kernel(w_hbm, idx_hbm, out_hbm, idx_a, idx_b, buf_a, buf_b,
                      sem_a, sem_b):
        wid = lax.axis_index("s") * 2 + lax.axis_index("c")
        base = wid * BPW
        idxv = [idx_a, idx_b]
        bufv = [buf_a, buf_b]
        semv = [sem_a, sem_b]
        copies = [None, None]
        pltpu.sync_copy(idx_hbm.at[wid, 0], idx_a)
        copies[0] = pltpu.async_copy(w_hbm.at[idx_a], buf_a, sem_a)
        for c in range(NCH):
            cur = c % 2
            nxt = (c + 1) % 2
            if c + 1 < NCH:
                pltpu.sync_copy(idx_hbm.at[wid, c + 1], idxv[nxt])
                copies[nxt] = pltpu.async_copy(w_hbm.at[idxv[nxt]],
                                               bufv[nxt], semv[nxt])
            copies[cur].wait()
            pltpu.sync_copy(bufv[cur], out_hbm.at[pl.ds(base + c * CH, CH)])

    return gather_kernel(w, idx3)


def _st_loss_body(x_ref, q_ref, st_ref, loss_ref):
    i = pl.program_id(0)
    x = x_ref[...]
    q = q_ref[...]
    diff = q - x
    st_ref[...] = x + diff
    part = jnp.sum(diff * diff)
    prev = jnp.where(i == 0, 0.0, loss_ref[0, 0])
    acc = prev + part
    scale = jnp.where(i == GRID - 1, jnp.float32(1.25 / (T * D)), 1.0)
    loss_ref[...] = jnp.reshape(acc * scale, (1, 1))


def _st_loss(xf, q):
    return pl.pallas_call(
        _st_loss_body,
        grid=(GRID,),
        in_specs=[
            pl.BlockSpec((TM, D), lambda i: (i, 0)),
            pl.BlockSpec((TM, D), lambda i: (i, 0)),
        ],
        out_specs=[
            pl.BlockSpec((TM, D), lambda i: (i, 0)),
            pl.BlockSpec((1, 1), lambda i: (0, 0)),
        ],
        out_shape=[
            jax.ShapeDtypeStruct((T, D), jnp.float32),
            jax.ShapeDtypeStruct((1, 1), jnp.float32),
        ],
    )(xf, q)


def kernel(x, W):
    B, S, _ = x.shape
    xf = x.reshape(T, D)
    x2 = jnp.sum(xf ** 2, axis=1, keepdims=True)
    w2r = jnp.sum(W ** 2, axis=1).reshape(1, K)
    idx = _dist_argmin(xf, W, x2, w2r)
    idx_flat = idx.reshape(T)
    q = _sc_gather(W, idx.reshape(NW, NCH, CH))
    st, loss2 = _st_loss(xf, q)
    return (st.reshape(B, S, D), loss2[0, 0], idx_flat.reshape(B, S))


# speedup vs baseline: 1.3751x; 1.1737x over previous
import functools

import jax
import jax.numpy as jnp
from jax import lax
from jax.experimental import pallas as pl
from jax.experimental.pallas import tpu as pltpu
from jax.experimental.pallas import tpu_sc as plsc

K = 8192
D = 256
T = 16384
TM = 512
TN = 2048
NCHUNK = K // TN
GRID = T // TM

NW = 32
BPW = T // NW
CH = 128
NCH = BPW // CH


def _dist_argmin_body(x_ref, wt_ref, x2_ref, w2_ref, idx_ref):
    x = x_ref[...]
    x2 = x2_ref[...]
    xb = x.astype(jnp.bfloat16)

    wb = wt_ref[...].astype(jnp.bfloat16)
    mm = lax.dot_general(xb, wb, (((1,), (1,)), ((), ())),
                         preferred_element_type=jnp.float32)
    d = (x2 + w2_ref[...]) - 2.0 * mm
    idx_ref[...] = jnp.argmin(d, axis=1).astype(jnp.int32).reshape(TM, 1)


def _dist_argmin(xf, w, x2, w2r):
    return pl.pallas_call(
        _dist_argmin_body,
        grid=(GRID,),
        in_specs=[
            pl.BlockSpec((TM, D), lambda i: (i, 0)),
            pl.BlockSpec((K, D), lambda i: (0, 0)),
            pl.BlockSpec((TM, 1), lambda i: (i, 0)),
            pl.BlockSpec((1, K), lambda i: (0, 0)),
        ],
        out_specs=pl.BlockSpec((TM, 1), lambda i: (i, 0)),
        out_shape=jax.ShapeDtypeStruct((T, 1), jnp.int32),
    )(xf, w, x2, w2r)


def _sc_gather(w, idx3):
    mesh = plsc.VectorSubcoreMesh(core_axis_name="c", subcore_axis_name="s")

    @functools.partial(
        pl.kernel,
        out_type=jax.ShapeDtypeStruct((T, D), jnp.float32),
        mesh=mesh,
        scratch_types=[
            pltpu.VMEM((CH,), jnp.int32),
            pltpu.VMEM((CH,), jnp.int32),
            pltpu.VMEM((CH, D), jnp.float32),
            pltpu.VMEM((CH, D), jnp.float32),
            pltpu.SemaphoreType.DMA,
            pltpu.SemaphoreType.DMA,
        ],
    )
    def gather_kernel(w_hbm, idx_hbm, out_hbm, idx_a, idx_b, buf_a, buf_b,
                      sem_a, sem_b):
        wid = lax.axis_index("s") * 2 + lax.axis_index("c")
        base = wid * BPW
        idxv = [idx_a, idx_b]
        bufv = [buf_a, buf_b]
        semv = [sem_a, sem_b]
        copies = [None, None]
        pltpu.sync_copy(idx_hbm.at[wid, 0], idx_a)
        copies[0] = pltpu.async_copy(w_hbm.at[idx_a], buf_a, sem_a)
        for c in range(NCH):
            cur = c % 2
            nxt = (c + 1) % 2
            if c + 1 < NCH:
                pltpu.sync_copy(idx_hbm.at[wid, c + 1], idxv[nxt])
                copies[nxt] = pltpu.async_copy(w_hbm.at[idxv[nxt]],
                                               bufv[nxt], semv[nxt])
            copies[cur].wait()
            pltpu.sync_copy(bufv[cur], out_hbm.at[pl.ds(base + c * CH, CH)])

    return gather_kernel(w, idx3)


def _st_loss_body(x_ref, q_ref, st_ref, loss_ref):
    i = pl.program_id(0)
    x = x_ref[...]
    q = q_ref[...]
    diff = q - x
    st_ref[...] = x + diff
    part = jnp.sum(diff * diff)
    prev = jnp.where(i == 0, 0.0, loss_ref[0, 0])
    acc = prev + part
    scale = jnp.where(i == GRID - 1, jnp.float32(1.25 / (T * D)), 1.0)
    loss_ref[...] = jnp.reshape(acc * scale, (1, 1))


def _st_loss(xf, q):
    return pl.pallas_call(
        _st_loss_body,
        grid=(GRID,),
        in_specs=[
            pl.BlockSpec((TM, D), lambda i: (i, 0)),
            pl.BlockSpec((TM, D), lambda i: (i, 0)),
        ],
        out_specs=[
            pl.BlockSpec((TM, D), lambda i: (i, 0)),
            pl.BlockSpec((1, 1), lambda i: (0, 0)),
        ],
        out_shape=[
            jax.ShapeDtypeStruct((T, D), jnp.float32),
            jax.ShapeDtypeStruct((1, 1), jnp.float32),
        ],
    )(xf, q)


def kernel(x, W):
    B, S, _ = x.shape
    xf = x.reshape(T, D)
    x2 = jnp.sum(xf ** 2, axis=1, keepdims=True)
    w2r = jnp.sum(W ** 2, axis=1).reshape(1, K)
    idx = _dist_argmin(xf, W, x2, w2r)
    idx_flat = idx.reshape(T)
    q = _sc_gather(W, idx.reshape(NW, NCH, CH))
    st, loss2 = _st_loss(xf, q)
    return (st.reshape(B, S, D), loss2[0, 0], idx_flat.reshape(B, S))


# speedup vs baseline: 1.4676x; 1.0672x over previous
import functools

import jax
import jax.numpy as jnp
from jax import lax
from jax.experimental import pallas as pl
from jax.experimental.pallas import tpu as pltpu
from jax.experimental.pallas import tpu_sc as plsc

K = 8192
D = 256
T = 16384
TM = 1024
TN = 2048
NCHUNK = K // TN
GRID = T // TM

NW = 32
BPW = T // NW
CH = 128
NCH = BPW // CH


def _dist_argmin_body(x_ref, wt_ref, x2_ref, w2_ref, idx_ref):
    x = x_ref[...]
    x2 = x2_ref[...]
    xb = x.astype(jnp.bfloat16)

    wb = wt_ref[...].astype(jnp.bfloat16)
    mm = lax.dot_general(xb, wb, (((1,), (1,)), ((), ())),
                         preferred_element_type=jnp.float32)
    d = (x2 + w2_ref[...]) - 2.0 * mm
    idx_ref[...] = jnp.argmin(d, axis=1).astype(jnp.int32).reshape(TM, 1)


def _dist_argmin(xf, w, x2, w2r):
    return pl.pallas_call(
        _dist_argmin_body,
        grid=(GRID,),
        in_specs=[
            pl.BlockSpec((TM, D), lambda i: (i, 0)),
            pl.BlockSpec((K, D), lambda i: (0, 0)),
            pl.BlockSpec((TM, 1), lambda i: (i, 0)),
            pl.BlockSpec((1, K), lambda i: (0, 0)),
        ],
        out_specs=pl.BlockSpec((TM, 1), lambda i: (i, 0)),
        out_shape=jax.ShapeDtypeStruct((T, 1), jnp.int32),
    )(xf, w, x2, w2r)


def _sc_gather(w, idx3):
    mesh = plsc.VectorSubcoreMesh(core_axis_name="c", subcore_axis_name="s")

    @functools.partial(
        pl.kernel,
        out_type=jax.ShapeDtypeStruct((T, D), jnp.float32),
        mesh=mesh,
        scratch_types=[
            pltpu.VMEM((CH,), jnp.int32),
            pltpu.VMEM((CH,), jnp.int32),
            pltpu.VMEM((CH, D), jnp.float32),
            pltpu.VMEM((CH, D), jnp.float32),
            pltpu.SemaphoreType.DMA,
            pltpu.SemaphoreType.DMA,
        ],
    )
    def gather_kernel(w_hbm, idx_hbm, out_hbm, idx_a, idx_b, buf_a, buf_b,
                      sem_a, sem_b):
        wid = lax.axis_index("s") * 2 + lax.axis_index("c")
        base = wid * BPW
        idxv = [idx_a, idx_b]
        bufv = [buf_a, buf_b]
        semv = [sem_a, sem_b]
        copies = [None, None]
        pltpu.sync_copy(idx_hbm.at[wid, 0], idx_a)
        copies[0] = pltpu.async_copy(w_hbm.at[idx_a], buf_a, sem_a)
        for c in range(NCH):
            cur = c % 2
            nxt = (c + 1) % 2
            if c + 1 < NCH:
                pltpu.sync_copy(idx_hbm.at[wid, c + 1], idxv[nxt])
                copies[nxt] = pltpu.async_copy(w_hbm.at[idxv[nxt]],
                                               bufv[nxt], semv[nxt])
            copies[cur].wait()
            pltpu.sync_copy(bufv[cur], out_hbm.at[pl.ds(base + c * CH, CH)])

    return gather_kernel(w, idx3)


def _st_loss_body(x_ref, q_ref, st_ref, loss_ref):
    i = pl.program_id(0)
    x = x_ref[...]
    q = q_ref[...]
    diff = q - x
    st_ref[...] = x + diff
    part = jnp.sum(diff * diff)
    prev = jnp.where(i == 0, 0.0, loss_ref[0, 0])
    acc = prev + part
    scale = jnp.where(i == GRID - 1, jnp.float32(1.25 / (T * D)), 1.0)
    loss_ref[...] = jnp.reshape(acc * scale, (1, 1))


def _st_loss(xf, q):
    return pl.pallas_call(
        _st_loss_body,
        grid=(GRID,),
        in_specs=[
            pl.BlockSpec((TM, D), lambda i: (i, 0)),
            pl.BlockSpec((TM, D), lambda i: (i, 0)),
        ],
        out_specs=[
            pl.BlockSpec((TM, D), lambda i: (i, 0)),
            pl.BlockSpec((1, 1), lambda i: (0, 0)),
        ],
        out_shape=[
            jax.ShapeDtypeStruct((T, D), jnp.float32),
            jax.ShapeDtypeStruct((1, 1), jnp.float32),
        ],
    )(xf, q)


def kernel(x, W):
    B, S, _ = x.shape
    xf = x.reshape(T, D)
    x2 = jnp.sum(xf ** 2, axis=1, keepdims=True)
    w2r = jnp.sum(W ** 2, axis=1).reshape(1, K)
    idx = _dist_argmin(xf, W, x2, w2r)
    idx_flat = idx.reshape(T)
    q = _sc_gather(W, idx.reshape(NW, NCH, CH))
    st, loss2 = _st_loss(xf, q)
    return (st.reshape(B, S, D), loss2[0, 0], idx_flat.reshape(B, S))
